# baseline (speedup 1.0000x reference)
import jax
import jax.numpy as jnp
from jax.experimental import pallas as pl
from jax.experimental.pallas import tpu as pltpu

_BB = 128


def _body(xc_ref, xn_ref, ef_ref, ws_ref, wv_ref, we_ref, bvr_ref, bsr_ref,
          out_ref):
    bb, n, d = xn_ref.shape
    c = ws_ref.shape[1]

    xcv = xc_ref[...].reshape(bb, d)
    vcb = jnp.dot(xcv, wv_ref[...], preferred_element_type=jnp.float32)
    vcb = vcb + bvr_ref[...]

    center = jnp.dot(xcv, ws_ref[...], preferred_element_type=jnp.float32)
    center = center + bsr_ref[...]

    xn = xn_ref[...].reshape(bb * n, d)
    ef = ef_ref[...].reshape(bb * n, d)
    nbr = jnp.dot(xn, ws_ref[...], preferred_element_type=jnp.float32)
    nbr = nbr + jnp.dot(ef, we_ref[...], preferred_element_type=jnp.float32)
    nbr = nbr.reshape(bb, n, c) + vcb[:, None, :] + bsr_ref[...][None]

    block = jnp.concatenate([center[:, None, :], nbr], axis=1)
    out_ref[...] = block.reshape(bb * (n + 1), c)


def kernel(central_node_features, neighbor_node_features, edge_features,
           Wq, bq, Wk, bk, Wv, bv, We, Wskip, bskip):
    b, n, d = neighbor_node_features.shape
    c = Wskip.shape[0]
    m = b * (n + 1)

    ws_t = Wskip.T
    wv_t = Wv.T
    we_t = We.T
    bvr = bv.reshape(1, c)
    bsr = bskip.reshape(1, c)

    out = pl.pallas_call(
        _body,
        grid=(b // _BB,),
        in_specs=[
            pl.BlockSpec((_BB, 1, d), lambda i: (i, 0, 0)),
            pl.BlockSpec((_BB, n, d), lambda i: (i, 0, 0)),
            pl.BlockSpec((_BB, n, d), lambda i: (i, 0, 0)),
            pl.BlockSpec((d, c), lambda i: (0, 0)),
            pl.BlockSpec((d, c), lambda i: (0, 0)),
            pl.BlockSpec((d, c), lambda i: (0, 0)),
            pl.BlockSpec((1, c), lambda i: (0, 0)),
            pl.BlockSpec((1, c), lambda i: (0, 0)),
        ],
        out_specs=pl.BlockSpec((_BB * (n + 1), c), lambda i: (i, 0)),
        out_shape=jax.ShapeDtypeStruct((m, c), jnp.float32),
        compiler_params=pltpu.CompilerParams(
            dimension_semantics=("arbitrary",)),
    )(central_node_features, neighbor_node_features, edge_features,
      ws_t, wv_t, we_t, bvr, bsr)
    return out


# speedup vs baseline: 1.0452x; 1.0452x over previous
import jax
import jax.numpy as jnp
from jax.experimental import pallas as pl
from jax.experimental.pallas import tpu as pltpu

_BB = 128
_NP = 56


def _body(xc_ref, xn_ref, ef_ref, ws_ref, wv_ref, we_ref, bvr_ref, bsr_ref,
          s_ref, out_ref, xn_pad, ef_pad, out_scr, sem_in, sem_out):
    n = xn_ref.shape[1]
    d = xn_ref.shape[2]
    c = ws_ref.shape[1]
    rows = _BB * _NP

    cin = pltpu.make_async_copy(xc_ref, xn_pad.at[:, 0:1, :], sem_in)
    nin = pltpu.make_async_copy(xn_ref, xn_pad.at[:, 1:n + 1, :], sem_in)
    ein = pltpu.make_async_copy(ef_ref, ef_pad.at[:, 1:n + 1, :], sem_in)
    cin.start()
    nin.start()
    ein.start()
    ef_pad[:, 0:1, :] = jnp.zeros((_BB, 1, d), jnp.float32)
    cin.wait()
    nin.wait()
    ein.wait()

    xnp = xn_pad[...].reshape(rows, d)
    efp = ef_pad[...].reshape(rows, d)
    xcv = xc_ref[...].reshape(_BB, d)

    vcb = jnp.dot(xcv, wv_ref[...], preferred_element_type=jnp.float32)
    vcb = vcb + bvr_ref[...]

    out_val = jnp.dot(xnp, ws_ref[...], preferred_element_type=jnp.float32)
    out_val = out_val + jnp.dot(efp, we_ref[...],
                                preferred_element_type=jnp.float32)
    out_val = out_val + jnp.dot(s_ref[...], vcb,
                                preferred_element_type=jnp.float32)
    out_val = out_val + bsr_ref[...]
    out_scr[...] = out_val.reshape(_BB, _NP, c)

    copies = [
        pltpu.make_async_copy(out_scr.at[s, 0:n + 1, :],
                              out_ref.at[pl.ds((n + 1) * s, n + 1), :],
                              sem_out)
        for s in range(_BB)
    ]
    for cp in copies:
        cp.start()
    for cp in copies:
        cp.wait()


def kernel(central_node_features, neighbor_node_features, edge_features,
           Wq, bq, Wk, bk, Wv, bv, We, Wskip, bskip):
    b, n, d = neighbor_node_features.shape
    c = Wskip.shape[0]
    m = b * (n + 1)
    rows = _BB * _NP

    ws_t = Wskip.T
    wv_t = Wv.T
    we_t = We.T
    bvr = (bv).reshape(1, c)
    bsr = bskip.reshape(1, c)
    t = jnp.arange(rows) % _NP
    sel = (((jnp.arange(rows) // _NP) == jnp.arange(_BB)[:, None]).T
           & (t >= 1)[:, None] & (t <= n)[:, None]).astype(jnp.float32)

    out = pl.pallas_call(
        _body,
        grid=(b // _BB,),
        in_specs=[
            pl.BlockSpec((_BB, 1, d), lambda i: (i, 0, 0)),
            pl.BlockSpec((_BB, n, d), lambda i: (i, 0, 0)),
            pl.BlockSpec((_BB, n, d), lambda i: (i, 0, 0)),
            pl.BlockSpec((d, c), lambda i: (0, 0)),
            pl.BlockSpec((d, c), lambda i: (0, 0)),
            pl.BlockSpec((d, c), lambda i: (0, 0)),
            pl.BlockSpec((1, c), lambda i: (0, 0)),
            pl.BlockSpec((1, c), lambda i: (0, 0)),
            pl.BlockSpec((rows, _BB), lambda i: (0, 0)),
        ],
        out_specs=pl.BlockSpec((_BB * (n + 1), c), lambda i: (i, 0)),
        out_shape=jax.ShapeDtypeStruct((m, c), jnp.float32),
        scratch_shapes=[
            pltpu.VMEM((_BB, _NP, d), jnp.float32),
            pltpu.VMEM((_BB, _NP, d), jnp.float32),
            pltpu.VMEM((_BB, _NP, c), jnp.float32),
            pltpu.SemaphoreType.DMA,
            pltpu.SemaphoreType.DMA,
        ],
        compiler_params=pltpu.CompilerParams(
            dimension_semantics=("arbitrary",)),
    )(central_node_features, neighbor_node_features, edge_features,
      ws_t, wv_t, we_t, bvr, bsr, sel)
    return out


# speedup vs baseline: 1.2441x; 1.1902x over previous
import jax
import jax.numpy as jnp
from jax.experimental import pallas as pl
from jax.experimental.pallas import tpu as pltpu

_BB = 128
_NP = 56


def _body(xc_ref, xn_ref, ef_ref, ws_ref, wv_ref, we_ref, bvr_ref, bsr_ref,
          s_ref, out_ref, xn_pad, ef_pad, out_scr, sem_in, sem_out):
    n = xn_ref.shape[1]
    d = xn_ref.shape[2]
    c = ws_ref.shape[1]
    rows = _BB * _NP

    cin = pltpu.make_async_copy(xc_ref, xn_pad.at[:, 0:1, :], sem_in)
    nin = pltpu.make_async_copy(xn_ref, xn_pad.at[:, 1:n + 1, :], sem_in)
    ein = pltpu.make_async_copy(ef_ref, ef_pad.at[:, 1:n + 1, :], sem_in)
    cin.start()
    nin.start()
    ein.start()
    ef_pad[:, 0:1, :] = jnp.zeros((_BB, 1, d), jnp.float32)
    cin.wait()
    nin.wait()
    ein.wait()

    xnp = xn_pad[...].reshape(rows, d)
    efp = ef_pad[...].reshape(rows, d)
    xcv = xc_ref[...].reshape(_BB, d)

    vcb = jnp.dot(xcv, wv_ref[...], preferred_element_type=jnp.float32)
    vcb = vcb + bvr_ref[...]

    out_val = jnp.dot(xnp, ws_ref[...], preferred_element_type=jnp.float32)
    out_val = out_val + jnp.dot(efp, we_ref[...],
                                preferred_element_type=jnp.float32)
    out_val = out_val + jnp.dot(s_ref[...], vcb,
                                preferred_element_type=jnp.float32)
    out_val = out_val + bsr_ref[...]
    out_scr[...] = out_val.reshape(_BB, _NP, c)

    j = pl.program_id(0)
    base = j * _BB * (n + 1)
    copies = [
        pltpu.make_async_copy(out_scr.at[s, 0:n + 1, :],
                              out_ref.at[pl.ds(base + (n + 1) * s, n + 1), :],
                              sem_out)
        for s in range(_BB)
    ]
    for cp in copies:
        cp.start()
    for cp in copies:
        cp.wait()


def kernel(central_node_features, neighbor_node_features, edge_features,
           Wq, bq, Wk, bk, Wv, bv, We, Wskip, bskip):
    b, n, d = neighbor_node_features.shape
    c = Wskip.shape[0]
    m = b * (n + 1)
    rows = _BB * _NP

    ws_t = Wskip.T
    wv_t = Wv.T
    we_t = We.T
    bvr = (bv).reshape(1, c)
    bsr = bskip.reshape(1, c)
    t = jnp.arange(rows) % _NP
    sel = (((jnp.arange(rows) // _NP) == jnp.arange(_BB)[:, None]).T
           & (t >= 1)[:, None] & (t <= n)[:, None]).astype(jnp.float32)

    out = pl.pallas_call(
        _body,
        grid=(b // _BB,),
        in_specs=[
            pl.BlockSpec((_BB, 1, d), lambda i: (i, 0, 0)),
            pl.BlockSpec((_BB, n, d), lambda i: (i, 0, 0)),
            pl.BlockSpec((_BB, n, d), lambda i: (i, 0, 0)),
            pl.BlockSpec((d, c), lambda i: (0, 0)),
            pl.BlockSpec((d, c), lambda i: (0, 0)),
            pl.BlockSpec((d, c), lambda i: (0, 0)),
            pl.BlockSpec((1, c), lambda i: (0, 0)),
            pl.BlockSpec((1, c), lambda i: (0, 0)),
            pl.BlockSpec((rows, _BB), lambda i: (0, 0)),
        ],
        out_specs=pl.BlockSpec(memory_space=pltpu.MemorySpace.HBM),
        out_shape=jax.ShapeDtypeStruct((m, c), jnp.float32),
        scratch_shapes=[
            pltpu.VMEM((_BB, _NP, d), jnp.float32),
            pltpu.VMEM((_BB, _NP, d), jnp.float32),
            pltpu.VMEM((_BB, _NP, c), jnp.float32),
            pltpu.SemaphoreType.DMA,
            pltpu.SemaphoreType.DMA,
        ],
        compiler_params=pltpu.CompilerParams(
            dimension_semantics=("arbitrary",)),
    )(central_node_features, neighbor_node_features, edge_features,
      ws_t, wv_t, we_t, bvr, bsr, sel)
    return out


# speedup vs baseline: 1.2791x; 1.0281x over previous
import jax
import jax.numpy as jnp
from jax.experimental import pallas as pl
from jax.experimental.pallas import tpu as pltpu

_BB = 128
_NP = 56


def _body(xc_ref, xn_ref, ef_ref, ws_ref, wv_ref, we_ref, bvr_ref, bsr_ref,
          s_ref, out_ref, xn_pad, ef_pad, out_scr, sem_in, sem_out):
    j = pl.program_id(0)
    nsteps = pl.num_programs(0)
    n = xn_ref.shape[1]
    d = xn_ref.shape[2]
    c = ws_ref.shape[1]
    rows = _BB * _NP
    ib = j % 2

    @pl.when(j < nsteps - 1)
    def _start_loads():
        ef_pad[ib, :, 0:1, :] = jnp.zeros((_BB, 1, d), jnp.float32)
        base = j * _BB
        pltpu.make_async_copy(xc_ref.at[pl.ds(base, _BB), :, :],
                              xn_pad.at[ib, :, 0:1, :],
                              sem_in.at[ib]).start()
        pltpu.make_async_copy(xn_ref.at[pl.ds(base, _BB), :, :],
                              xn_pad.at[ib, :, 1:n + 1, :],
                              sem_in.at[ib]).start()
        pltpu.make_async_copy(ef_ref.at[pl.ds(base, _BB), :, :],
                              ef_pad.at[ib, :, 1:n + 1, :],
                              sem_in.at[ib]).start()

    @pl.when(j > 0)
    def _compute_prev():
        pb = 1 - ib
        base = (j - 1) * _BB
        pltpu.make_async_copy(xc_ref.at[pl.ds(base, _BB), :, :],
                              xn_pad.at[pb, :, 0:1, :],
                              sem_in.at[pb]).wait()
        pltpu.make_async_copy(xn_ref.at[pl.ds(base, _BB), :, :],
                              xn_pad.at[pb, :, 1:n + 1, :],
                              sem_in.at[pb]).wait()
        pltpu.make_async_copy(ef_ref.at[pl.ds(base, _BB), :, :],
                              ef_pad.at[pb, :, 1:n + 1, :],
                              sem_in.at[pb]).wait()

        xnp = xn_pad[pb].reshape(rows, d)
        efp = ef_pad[pb].reshape(rows, d)
        xcv = xn_pad[pb, :, 0, :]

        vcb = jnp.dot(xcv, wv_ref[...], preferred_element_type=jnp.float32)
        vcb = vcb + bvr_ref[...]

        out_val = jnp.dot(xnp, ws_ref[...], preferred_element_type=jnp.float32)
        out_val = out_val + jnp.dot(efp, we_ref[...],
                                    preferred_element_type=jnp.float32)
        out_val = out_val + jnp.dot(s_ref[...], vcb,
                                    preferred_element_type=jnp.float32)
        out_val = out_val + bsr_ref[...]
        out_scr[...] = out_val.reshape(_BB, _NP, c)

        obase = base * (n + 1)
        copies = [
            pltpu.make_async_copy(
                out_scr.at[s, 0:n + 1, :],
                out_ref.at[pl.ds(obase + (n + 1) * s, n + 1), :],
                sem_out)
            for s in range(_BB)
        ]
        for cp in copies:
            cp.start()
        for cp in copies:
            cp.wait()


def kernel(central_node_features, neighbor_node_features, edge_features,
           Wq, bq, Wk, bk, Wv, bv, We, Wskip, bskip):
    b, n, d = neighbor_node_features.shape
    c = Wskip.shape[0]
    m = b * (n + 1)
    rows = _BB * _NP
    g = b // _BB

    ws_t = Wskip.T
    wv_t = Wv.T
    we_t = We.T
    bvr = bv.reshape(1, c)
    bsr = bskip.reshape(1, c)
    t = jnp.arange(rows) % _NP
    sel = (((jnp.arange(rows) // _NP) == jnp.arange(_BB)[:, None]).T
           & (t >= 1)[:, None] & (t <= n)[:, None]).astype(jnp.float32)

    hbm = pl.BlockSpec(memory_space=pltpu.MemorySpace.HBM)
    out = pl.pallas_call(
        _body,
        grid=(g + 1,),
        in_specs=[
            hbm,
            hbm,
            hbm,
            pl.BlockSpec((d, c), lambda i: (0, 0)),
            pl.BlockSpec((d, c), lambda i: (0, 0)),
            pl.BlockSpec((d, c), lambda i: (0, 0)),
            pl.BlockSpec((1, c), lambda i: (0, 0)),
            pl.BlockSpec((1, c), lambda i: (0, 0)),
            pl.BlockSpec((rows, _BB), lambda i: (0, 0)),
        ],
        out_specs=pl.BlockSpec(memory_space=pltpu.MemorySpace.HBM),
        out_shape=jax.ShapeDtypeStruct((m, c), jnp.float32),
        scratch_shapes=[
            pltpu.VMEM((2, _BB, _NP, d), jnp.float32),
            pltpu.VMEM((2, _BB, _NP, d), jnp.float32),
            pltpu.VMEM((_BB, _NP, c), jnp.float32),
            pltpu.SemaphoreType.DMA((2,)),
            pltpu.SemaphoreType.DMA,
        ],
        compiler_params=pltpu.CompilerParams(
            dimension_semantics=("arbitrary",)),
    )(central_node_features, neighbor_node_features, edge_features,
      ws_t, wv_t, we_t, bvr, bsr, sel)
    return out
